# baseline (speedup 1.0000x reference)
import functools

import jax
import jax.numpy as jnp
from jax import lax
from jax.experimental import pallas as pl
from jax.experimental.pallas import tpu as pltpu
from jax.experimental.pallas import tpu_sc as plsc

_H = 16


def _qkv_kernel(x_ref, w_ref, qkv_ref, sc_ref, *, H):
    xb = x_ref[...].astype(jnp.bfloat16)
    wb = w_ref[...].astype(jnp.bfloat16)
    acc = jax.lax.dot_general(
        xb, wb, (((1,), (1,)), ((), ())), preferred_element_type=jnp.float32
    )
    TM, C3 = acc.shape
    Ch = C3 // (3 * H)
    sq = acc[:, : C3 // 3] ** 2
    s = jnp.sqrt(sq.reshape(TM, H, Ch).sum(axis=2))
    sc_ref[...] = s.T
    qkv_ref[...] = acc.astype(jnp.bfloat16)


def _thresh_kernel(sc_ref, gk_ref, gv_ref, dst_ref, *, keep, H, N, nchunk):
    s = sc_ref[...]
    BH = s.shape[0]
    si = lax.bitcast_convert_type(s, jnp.int32)

    def body(_, lohi):
        lo, hi = lohi
        mid = lo + (hi - lo + 1) // 2
        cnt = jnp.sum((si >= mid).astype(jnp.int32), axis=1, keepdims=True)
        ok = cnt >= keep
        return jnp.where(ok, mid, lo), jnp.where(ok, hi, mid - 1)

    lo = jnp.zeros((BH, 1), jnp.int32)
    hi = jnp.full((BH, 1), 0x7F7FFFFF, jnp.int32)
    lo, _ = lax.fori_loop(0, 31, body, (lo, hi))
    gt = (si > lo).astype(jnp.int32)
    eq = (si == lo).astype(jnp.int32)
    budget = keep - jnp.sum(gt, axis=1, keepdims=True)
    eq_rank = eq
    sh = 1
    while sh < N:
        pad = jnp.zeros((BH, sh), jnp.int32)
        eq_rank = eq_rank + jnp.concatenate([pad, eq_rank[:, : N - sh]], axis=1)
        sh *= 2
    take = gt + eq * (eq_rank <= budget).astype(jnp.int32)
    pref = take
    sh = 1
    while sh < N:
        pad = jnp.zeros((BH, sh), jnp.int32)
        pref = pref + jnp.concatenate([pad, pref[:, : N - sh]], axis=1)
        sh *= 2
    pos = pref - take
    w2 = lax.broadcasted_iota(jnp.int32, (BH, N), 0)
    tok = lax.broadcasted_iota(jnp.int32, (BH, N), 1)
    krow = ((w2 // H) * N + tok) * nchunk + H + (w2 % H)
    gk_ref[...] = krow
    gv_ref[...] = krow + H
    dst_ref[...] = w2 * (keep + 1) + jnp.where(take > 0, pos, keep)


def _sc_body(gk_hbm, gv_hbm, dst_hbm, tab_hbm, kout_hbm, vout_hbm,
             gidx_v, didx_v, rows_v, semg, sems, *, N):
    w = lax.axis_index("s") * 2 + lax.axis_index("c")
    pltpu.sync_copy(dst_hbm.at[w], didx_v)
    pltpu.sync_copy(gk_hbm.at[w], gidx_v)
    pltpu.async_copy(tab_hbm.at[gidx_v], rows_v, semg).wait()
    pltpu.async_copy(rows_v, kout_hbm.at[didx_v], sems).wait()
    pltpu.sync_copy(gv_hbm.at[w], gidx_v)
    pltpu.async_copy(tab_hbm.at[gidx_v], rows_v, semg).wait()
    pltpu.async_copy(rows_v, vout_hbm.at[didx_v], sems).wait()


def _attn_kernel(q_ref, k_ref, v_ref, o_ref, *, scale, Ch):
    outs = []
    for hh in range(2):
        q = q_ref[:, hh * Ch:(hh + 1) * Ch]
        k = k_ref[hh]
        logits = jax.lax.dot_general(
            q, k, (((1,), (1,)), ((), ())), preferred_element_type=jnp.float32
        ) * scale
        logits = jnp.clip(logits, -50.0, 50.0)
        m = jnp.max(logits, axis=1, keepdims=True)
        p = jnp.exp(logits - m)
        wgt = (p / jnp.sum(p, axis=1, keepdims=True)).astype(jnp.bfloat16)
        outs.append(
            jax.lax.dot_general(
                wgt, v_ref[hh], (((1,), (0,)), ((), ())),
                preferred_element_type=jnp.float32,
            ).astype(jnp.bfloat16)
        )
    o_ref[...] = jnp.concatenate(outs, axis=1)


def _proj_kernel(a_ref, w_ref, b_ref, o_ref):
    wb = w_ref[...].astype(jnp.bfloat16)
    acc = jax.lax.dot_general(
        a_ref[...], wb, (((1,), (1,)), ((), ())),
        preferred_element_type=jnp.float32,
    )
    o_ref[...] = jnp.clip(acc + b_ref[...], -10.0, 10.0)


def kernel(x, W_qkv, W_proj, b_proj):
    B, N, C = x.shape
    H = _H
    Ch = C // H
    BN = B * N
    BH = B * H
    keep = N // 2
    scale = Ch ** (-0.5)
    nchunk = 3 * C // Ch

    x2 = x.reshape(BN, C)
    b2 = b_proj.reshape(1, C)

    TM = min(512, N)
    nrow = BN // TM
    ntile_b = N // TM

    qkv_flat, scores = pl.pallas_call(
        functools.partial(_qkv_kernel, H=H),
        grid=(nrow,),
        in_specs=[
            pl.BlockSpec((TM, C), lambda g: (g, 0)),
            pl.BlockSpec((3 * C, C), lambda g: (0, 0)),
        ],
        out_specs=[
            pl.BlockSpec((TM, 3 * C), lambda g: (g, 0)),
            pl.BlockSpec((H, TM), lambda g: (g // ntile_b, g % ntile_b)),
        ],
        out_shape=[
            jax.ShapeDtypeStruct((BN, 3 * C), jnp.bfloat16),
            jax.ShapeDtypeStruct((BH, N), jnp.float32),
        ],
    )(x2, W_qkv)

    gk, gv, dst = pl.pallas_call(
        functools.partial(_thresh_kernel, keep=keep, H=H, N=N, nchunk=nchunk),
        in_specs=[pl.BlockSpec((BH, N), lambda: (0, 0))],
        out_specs=[
            pl.BlockSpec((BH, N), lambda: (0, 0)),
            pl.BlockSpec((BH, N), lambda: (0, 0)),
            pl.BlockSpec((BH, N), lambda: (0, 0)),
        ],
        out_shape=[
            jax.ShapeDtypeStruct((BH, N), jnp.int32),
            jax.ShapeDtypeStruct((BH, N), jnp.int32),
            jax.ShapeDtypeStruct((BH, N), jnp.int32),
        ],
    )(scores)

    table = lax.bitcast_convert_type(
        qkv_flat.reshape(BN * nchunk, Ch // 2, 2), jnp.int32
    )

    mesh = plsc.VectorSubcoreMesh(core_axis_name="c", subcore_axis_name="s")
    kout, vout = pl.kernel(
        functools.partial(_sc_body, N=N),
        mesh=mesh,
        compiler_params=pltpu.CompilerParams(use_tc_tiling_on_sc=False),
        out_type=[
            jax.ShapeDtypeStruct((BH * (keep + 1), Ch // 2), jnp.int32),
            jax.ShapeDtypeStruct((BH * (keep + 1), Ch // 2), jnp.int32),
        ],
        scratch_types=[
            pltpu.VMEM((N,), jnp.int32),
            pltpu.VMEM((N,), jnp.int32),
            pltpu.VMEM((N, Ch // 2), jnp.int32),
            pltpu.SemaphoreType.DMA,
            pltpu.SemaphoreType.DMA,
        ],
    )(gk, gv, dst, table)

    ksel = lax.bitcast_convert_type(
        kout, jnp.bfloat16).reshape(BH, keep + 1, Ch)
    vsel = lax.bitcast_convert_type(
        vout, jnp.bfloat16).reshape(BH, keep + 1, Ch)

    TMq = min(512, N)
    nq = N // TMq
    H2 = H // 2

    attn_out = pl.pallas_call(
        functools.partial(_attn_kernel, scale=scale, Ch=Ch),
        grid=(B, H2, nq),
        in_specs=[
            pl.BlockSpec((TMq, 2 * Ch), lambda b, h2, qt: (b * nq + qt, h2)),
            pl.BlockSpec((2, keep, Ch), lambda b, h2, qt: (b * H2 + h2, 0, 0)),
            pl.BlockSpec((2, keep, Ch), lambda b, h2, qt: (b * H2 + h2, 0, 0)),
        ],
        out_specs=pl.BlockSpec(
            (TMq, 2 * Ch), lambda b, h2, qt: (b * nq + qt, h2)
        ),
        out_shape=jax.ShapeDtypeStruct((BN, C), jnp.bfloat16),
    )(qkv_flat, ksel, vsel)

    out = pl.pallas_call(
        _proj_kernel,
        grid=(nrow,),
        in_specs=[
            pl.BlockSpec((TM, C), lambda g: (g, 0)),
            pl.BlockSpec((C, C), lambda g: (0, 0)),
            pl.BlockSpec((1, C), lambda g: (0, 0)),
        ],
        out_specs=pl.BlockSpec((TM, C), lambda g: (g, 0)),
        out_shape=jax.ShapeDtypeStruct((BN, C), jnp.float32),
    )(attn_out, W_proj, b2)

    return out.reshape(B, N, C)


# speedup vs baseline: 28.9861x; 28.9861x over previous
import functools

import jax
import jax.numpy as jnp
from jax.experimental import pallas as pl

_H = 16
_NEG = -1e30


def _qkv_kernel(x_ref, w_ref, qkv_ref, sc_ref, *, H):
    xb = x_ref[...].astype(jnp.bfloat16)
    acc = jax.lax.dot_general(
        xb, w_ref[...], (((1,), (1,)), ((), ())),
        preferred_element_type=jnp.float32,
    )
    TM, C3 = acc.shape
    Ch = C3 // (3 * H)
    sq = acc[:, : C3 // 3] ** 2
    s = jnp.sqrt(sq.reshape(TM, H, Ch).sum(axis=2))
    sc_ref[...] = s.T[:, None, :]
    qkv_ref[...] = acc.astype(jnp.bfloat16)


def _thresh_kernel(sc_ref, bias_ref, *, keep):
    s = sc_ref[...].reshape(sc_ref.shape[0], sc_ref.shape[2])
    si = jax.lax.bitcast_convert_type(s, jnp.int32)

    def body(_, lohi):
        lo, hi = lohi
        mid = lo + (hi - lo + 1) // 2
        cnt = jnp.sum((si >= mid).astype(jnp.int32), axis=1, keepdims=True)
        ok = cnt >= keep
        return jnp.where(ok, mid, lo), jnp.where(ok, hi, mid - 1)

    lo = jnp.zeros((si.shape[0], 1), jnp.int32)
    hi = jnp.full((si.shape[0], 1), 0x7F7FFFFF, jnp.int32)
    lo, _ = jax.lax.fori_loop(0, 31, body, (lo, hi))
    bias = jnp.where(si >= lo, 0.0, _NEG).astype(jnp.float32)
    bias_ref[...] = bias[:, None, :]


def _attn_kernel(q_ref, k_ref, v_ref, bias_ref, o_ref, *, scale, Ch):
    outs = []
    for hh in range(2):
        sl = slice(hh * Ch, (hh + 1) * Ch)
        q = q_ref[:, sl]
        k = k_ref[:, sl]
        logits = jax.lax.dot_general(
            q, k, (((1,), (1,)), ((), ())), preferred_element_type=jnp.float32
        ) * scale
        logits = jnp.clip(logits, -50.0, 50.0) + bias_ref[hh]
        p = jnp.exp(logits)
        w = (p * (1.0 / jnp.sum(p, axis=1, keepdims=True))).astype(jnp.bfloat16)
        outs.append(
            jax.lax.dot_general(
                w, v_ref[:, sl], (((1,), (0,)), ((), ())),
                preferred_element_type=jnp.float32,
            ).astype(jnp.bfloat16)
        )
    o_ref[...] = jnp.concatenate(outs, axis=1)


def _proj_kernel(a_ref, w_ref, b_ref, o_ref):
    acc = jax.lax.dot_general(
        a_ref[...], w_ref[...], (((1,), (1,)), ((), ())),
        preferred_element_type=jnp.float32,
    )
    o_ref[...] = jnp.clip(acc + b_ref[...], -10.0, 10.0)


def kernel(x, W_qkv, W_proj, b_proj):
    B, N, C = x.shape
    H = _H
    Ch = C // H
    BN = B * N
    keep = N // 2
    scale = Ch ** (-0.5)

    x2 = x.reshape(BN, C)
    wq_bf = W_qkv.astype(jnp.bfloat16)
    wp_bf = W_proj.astype(jnp.bfloat16)
    b2 = b_proj.reshape(1, C)

    TM = min(512, N)
    nrow = BN // TM
    ntile_b = N // TM

    qkv_flat, scores = pl.pallas_call(
        functools.partial(_qkv_kernel, H=H),
        grid=(nrow,),
        in_specs=[
            pl.BlockSpec((TM, C), lambda g: (g, 0)),
            pl.BlockSpec((3 * C, C), lambda g: (0, 0)),
        ],
        out_specs=[
            pl.BlockSpec((TM, 3 * C), lambda g: (g, 0)),
            pl.BlockSpec((H, 1, TM), lambda g: (g // ntile_b, 0, g % ntile_b)),
        ],
        out_shape=[
            jax.ShapeDtypeStruct((BN, 3 * C), jnp.bfloat16),
            jax.ShapeDtypeStruct((B * H, 1, N), jnp.float32),
        ],
    )(x2, wq_bf)

    bias = pl.pallas_call(
        functools.partial(_thresh_kernel, keep=keep),
        in_specs=[pl.BlockSpec((B * H, 1, N), lambda: (0, 0, 0))],
        out_specs=pl.BlockSpec((B * H, 1, N), lambda: (0, 0, 0)),
        out_shape=jax.ShapeDtypeStruct((B * H, 1, N), jnp.float32),
    )(scores)

    TMq = min(1024, N)
    nq = N // TMq
    H2 = H // 2

    attn_out = pl.pallas_call(
        functools.partial(_attn_kernel, scale=scale, Ch=Ch),
        grid=(B, H2, nq),
        in_specs=[
            pl.BlockSpec((TMq, 2 * Ch), lambda b, h2, qt: (b * nq + qt, h2)),
            pl.BlockSpec((N, 2 * Ch), lambda b, h2, qt: (b, H2 + h2)),
            pl.BlockSpec((N, 2 * Ch), lambda b, h2, qt: (b, H + h2)),
            pl.BlockSpec((2, 1, N), lambda b, h2, qt: (b * H2 + h2, 0, 0)),
        ],
        out_specs=pl.BlockSpec(
            (TMq, 2 * Ch), lambda b, h2, qt: (b * nq + qt, h2)
        ),
        out_shape=jax.ShapeDtypeStruct((BN, C), jnp.bfloat16),
    )(qkv_flat, qkv_flat, qkv_flat, bias)

    out = pl.pallas_call(
        _proj_kernel,
        grid=(nrow,),
        in_specs=[
            pl.BlockSpec((TM, C), lambda g: (g, 0)),
            pl.BlockSpec((C, C), lambda g: (0, 0)),
            pl.BlockSpec((1, C), lambda g: (0, 0)),
        ],
        out_specs=pl.BlockSpec((TM, C), lambda g: (g, 0)),
        out_shape=jax.ShapeDtypeStruct((BN, C), jnp.float32),
    )(attn_out, wp_bf, b2)

    return out.reshape(B, N, C)


# speedup vs baseline: 33.1811x; 1.1447x over previous
import functools

import jax
import jax.numpy as jnp
from jax.experimental import pallas as pl

_H = 16
_NEG = -1e30


def _qkv_kernel(x_ref, w_ref, qkv_ref, sc_ref, *, H):
    xb = x_ref[...].astype(jnp.bfloat16)
    acc = jax.lax.dot_general(
        xb, w_ref[...], (((1,), (1,)), ((), ())),
        preferred_element_type=jnp.float32,
    )
    TM, C3 = acc.shape
    Ch = C3 // (3 * H)
    sq = acc[:, : C3 // 3] ** 2
    s = jnp.sqrt(sq.reshape(TM, H, Ch).sum(axis=2))
    sc_ref[...] = s.T[:, None, :]
    qkv_ref[...] = acc.astype(jnp.bfloat16)


def _thresh_kernel(sc_ref, bias_ref, *, keep):
    s = sc_ref[...].reshape(sc_ref.shape[0], sc_ref.shape[2])
    si = jax.lax.bitcast_convert_type(s, jnp.int32)

    def body(_, lohi):
        lo, hi = lohi
        mid = lo + (hi - lo + 1) // 2
        cnt = jnp.sum((si >= mid).astype(jnp.int32), axis=1, keepdims=True)
        ok = cnt >= keep
        return jnp.where(ok, mid, lo), jnp.where(ok, hi, mid - 1)

    lo = jnp.zeros((si.shape[0], 1), jnp.int32)
    hi = jnp.full((si.shape[0], 1), 0x7F7FFFFF, jnp.int32)
    lo, _ = jax.lax.fori_loop(0, 31, body, (lo, hi))
    bias = jnp.where(si >= lo, 0.0, _NEG).astype(jnp.float32)
    bias_ref[...] = bias[:, None, :]


def _attn_kernel(q_ref, k_ref, v_ref, bias_ref, o_ref, *, scale, Ch, HG):
    outs = []
    for hh in range(HG):
        sl = slice(hh * Ch, (hh + 1) * Ch)
        q = q_ref[:, sl]
        k = k_ref[:, sl]
        logits = jax.lax.dot_general(
            q, k, (((1,), (1,)), ((), ())), preferred_element_type=jnp.float32
        ) * scale
        logits = jnp.clip(logits, -50.0, 50.0) + bias_ref[hh]
        p = jnp.exp(logits)
        w = (p * (1.0 / jnp.sum(p, axis=1, keepdims=True))).astype(jnp.bfloat16)
        outs.append(
            jax.lax.dot_general(
                w, v_ref[:, sl], (((1,), (0,)), ((), ())),
                preferred_element_type=jnp.float32,
            ).astype(jnp.bfloat16)
        )
    o_ref[...] = jnp.concatenate(outs, axis=1)


def _proj_kernel(a_ref, w_ref, b_ref, o_ref):
    acc = jax.lax.dot_general(
        a_ref[...], w_ref[...], (((1,), (1,)), ((), ())),
        preferred_element_type=jnp.float32,
    )
    o_ref[...] = jnp.clip(acc + b_ref[...], -10.0, 10.0)


def kernel(x, W_qkv, W_proj, b_proj):
    B, N, C = x.shape
    H = _H
    Ch = C // H
    BN = B * N
    keep = N // 2
    scale = Ch ** (-0.5)

    x2 = x.reshape(BN, C)
    wq_bf = W_qkv.astype(jnp.bfloat16)
    wp_bf = W_proj.astype(jnp.bfloat16)
    b2 = b_proj.reshape(1, C)

    TM = min(1024, N)
    nrow = BN // TM
    ntile_b = N // TM

    qkv_flat, scores = pl.pallas_call(
        functools.partial(_qkv_kernel, H=H),
        grid=(nrow,),
        in_specs=[
            pl.BlockSpec((TM, C), lambda g: (g, 0)),
            pl.BlockSpec((3 * C, C), lambda g: (0, 0)),
        ],
        out_specs=[
            pl.BlockSpec((TM, 3 * C), lambda g: (g, 0)),
            pl.BlockSpec((H, 1, TM), lambda g: (g // ntile_b, 0, g % ntile_b)),
        ],
        out_shape=[
            jax.ShapeDtypeStruct((BN, 3 * C), jnp.bfloat16),
            jax.ShapeDtypeStruct((B * H, 1, N), jnp.float32),
        ],
    )(x2, wq_bf)

    bias = pl.pallas_call(
        functools.partial(_thresh_kernel, keep=keep),
        in_specs=[pl.BlockSpec((B * H, 1, N), lambda: (0, 0, 0))],
        out_specs=pl.BlockSpec((B * H, 1, N), lambda: (0, 0, 0)),
        out_shape=jax.ShapeDtypeStruct((B * H, 1, N), jnp.float32),
    )(scores)

    TMq = min(1024, N)
    nq = N // TMq
    HG = 4 if H % 4 == 0 else 2
    ng = H // HG
    ncolb = C // (HG * Ch)

    attn_out = pl.pallas_call(
        functools.partial(_attn_kernel, scale=scale, Ch=Ch, HG=HG),
        grid=(B, ng, nq),
        in_specs=[
            pl.BlockSpec((TMq, HG * Ch), lambda b, g, qt: (b * nq + qt, g)),
            pl.BlockSpec((N, HG * Ch), lambda b, g, qt: (b, ncolb + g)),
            pl.BlockSpec((N, HG * Ch), lambda b, g, qt: (b, 2 * ncolb + g)),
            pl.BlockSpec((HG, 1, N), lambda b, g, qt: (b * ng + g, 0, 0)),
        ],
        out_specs=pl.BlockSpec(
            (TMq, HG * Ch), lambda b, g, qt: (b * nq + qt, g)
        ),
        out_shape=jax.ShapeDtypeStruct((BN, C), jnp.bfloat16),
    )(qkv_flat, qkv_flat, qkv_flat, bias)

    out = pl.pallas_call(
        _proj_kernel,
        grid=(nrow,),
        in_specs=[
            pl.BlockSpec((TM, C), lambda g: (g, 0)),
            pl.BlockSpec((C, C), lambda g: (0, 0)),
            pl.BlockSpec((1, C), lambda g: (0, 0)),
        ],
        out_specs=pl.BlockSpec((TM, C), lambda g: (g, 0)),
        out_shape=jax.ShapeDtypeStruct((BN, C), jnp.float32),
    )(attn_out, wp_bf, b2)

    return out.reshape(B, N, C)


# speedup vs baseline: 35.4969x; 1.0698x over previous
import functools

import jax
import jax.numpy as jnp
from jax.experimental import pallas as pl

_H = 16
_NEG = -1e30


def _qkv_kernel(x_ref, w_ref, qkv_ref, sc_ref, *, H):
    xb = x_ref[...].astype(jnp.bfloat16)
    acc = jax.lax.dot_general(
        xb, w_ref[...], (((1,), (1,)), ((), ())),
        preferred_element_type=jnp.float32,
    )
    TM, C3 = acc.shape
    Ch = C3 // (3 * H)
    sq = acc[:, : C3 // 3] ** 2
    s = jnp.sqrt(sq.reshape(TM, H, Ch).sum(axis=2))
    sc_ref[...] = s.T[:, None, :]
    qkv_ref[...] = acc.astype(jnp.bfloat16)


def _thresh_kernel(sc_ref, bias_ref, *, keep):
    s = sc_ref[...].reshape(sc_ref.shape[0], sc_ref.shape[2])
    si = jax.lax.bitcast_convert_type(s, jnp.int32)

    def body(_, lohi):
        lo, hi = lohi
        mid = lo + (hi - lo + 1) // 2
        cnt = jnp.sum((si >= mid).astype(jnp.int32), axis=1, keepdims=True)
        ok = cnt >= keep
        return jnp.where(ok, mid, lo), jnp.where(ok, hi, mid - 1)

    lo = jnp.zeros((si.shape[0], 1), jnp.int32)
    hi = jnp.full((si.shape[0], 1), 0x7F7FFFFF, jnp.int32)
    lo, _ = jax.lax.fori_loop(0, 31, body, (lo, hi))
    bias = jnp.where(si >= lo, 0.0, _NEG).astype(jnp.float32)
    bias_ref[...] = bias[:, None, :]


def _attn_kernel(q_ref, k_ref, v_ref, bias_ref, o_ref, *, scale, Ch, HG):
    outs = []
    for hh in range(HG):
        sl = slice(hh * Ch, (hh + 1) * Ch)
        q = q_ref[:, sl]
        k = k_ref[:, sl]
        logits = jax.lax.dot_general(
            q, k, (((1,), (1,)), ((), ())), preferred_element_type=jnp.float32
        ) * scale
        logits = jnp.clip(logits, -50.0, 50.0) + bias_ref[hh]
        p = jnp.exp(logits)
        w = (p * (1.0 / jnp.sum(p, axis=1, keepdims=True))).astype(jnp.bfloat16)
        outs.append(
            jax.lax.dot_general(
                w, v_ref[:, sl], (((1,), (0,)), ((), ())),
                preferred_element_type=jnp.float32,
            ).astype(jnp.bfloat16)
        )
    o_ref[...] = jnp.concatenate(outs, axis=1)


def _proj_kernel(a_ref, w_ref, b_ref, o_ref):
    acc = jax.lax.dot_general(
        a_ref[...], w_ref[...], (((1,), (1,)), ((), ())),
        preferred_element_type=jnp.float32,
    )
    o_ref[...] = jnp.clip(acc + b_ref[...], -10.0, 10.0)


def kernel(x, W_qkv, W_proj, b_proj):
    B, N, C = x.shape
    H = _H
    Ch = C // H
    BN = B * N
    keep = N // 2
    scale = Ch ** (-0.5)

    x2 = x.reshape(BN, C)
    wq_bf = W_qkv.astype(jnp.bfloat16)
    wp_bf = W_proj.astype(jnp.bfloat16)
    b2 = b_proj.reshape(1, C)

    TM = min(1024, N)
    nrow = BN // TM
    ntile_b = N // TM

    qkv_flat, scores = pl.pallas_call(
        functools.partial(_qkv_kernel, H=H),
        grid=(nrow,),
        in_specs=[
            pl.BlockSpec((TM, C), lambda g: (g, 0)),
            pl.BlockSpec((3 * C, C), lambda g: (0, 0)),
        ],
        out_specs=[
            pl.BlockSpec((TM, 3 * C), lambda g: (g, 0)),
            pl.BlockSpec((H, 1, TM), lambda g: (g // ntile_b, 0, g % ntile_b)),
        ],
        out_shape=[
            jax.ShapeDtypeStruct((BN, 3 * C), jnp.bfloat16),
            jax.ShapeDtypeStruct((B * H, 1, N), jnp.float32),
        ],
    )(x2, wq_bf)

    bias = pl.pallas_call(
        functools.partial(_thresh_kernel, keep=keep),
        in_specs=[pl.BlockSpec((B * H, 1, N), lambda: (0, 0, 0))],
        out_specs=pl.BlockSpec((B * H, 1, N), lambda: (0, 0, 0)),
        out_shape=jax.ShapeDtypeStruct((B * H, 1, N), jnp.float32),
    )(scores)

    TMq = min(1024, N)
    nq = N // TMq
    HG = 8 if H % 8 == 0 else 2
    ng = H // HG
    ncolb = C // (HG * Ch)

    attn_out = pl.pallas_call(
        functools.partial(_attn_kernel, scale=scale, Ch=Ch, HG=HG),
        grid=(B, ng, nq),
        in_specs=[
            pl.BlockSpec((TMq, HG * Ch), lambda b, g, qt: (b * nq + qt, g)),
            pl.BlockSpec((N, HG * Ch), lambda b, g, qt: (b, ncolb + g)),
            pl.BlockSpec((N, HG * Ch), lambda b, g, qt: (b, 2 * ncolb + g)),
            pl.BlockSpec((HG, 1, N), lambda b, g, qt: (b * ng + g, 0, 0)),
        ],
        out_specs=pl.BlockSpec(
            (TMq, HG * Ch), lambda b, g, qt: (b * nq + qt, g)
        ),
        out_shape=jax.ShapeDtypeStruct((BN, C), jnp.bfloat16),
    )(qkv_flat, qkv_flat, qkv_flat, bias)

    out = pl.pallas_call(
        _proj_kernel,
        grid=(nrow,),
        in_specs=[
            pl.BlockSpec((TM, C), lambda g: (g, 0)),
            pl.BlockSpec((C, C), lambda g: (0, 0)),
            pl.BlockSpec((1, C), lambda g: (0, 0)),
        ],
        out_specs=pl.BlockSpec((TM, C), lambda g: (g, 0)),
        out_shape=jax.ShapeDtypeStruct((BN, C), jnp.float32),
    )(attn_out, wp_bf, b2)

    return out.reshape(B, N, C)
